# baseline (speedup 1.0000x reference)
import functools

import jax
import jax.numpy as jnp
from jax import lax
from jax.experimental import pallas as pl
from jax.experimental.pallas import tpu as pltpu
from jax.experimental.pallas import tpu_sc as plsc

_B = 4
_S = 4096
_D = 128
_N = _B * _S


@functools.cache
def _gather_kernel():
    info = plsc.get_sparse_core_info()
    nw = info.num_cores * info.num_subcores
    per_w = _N // nw
    w_per_b = _S // per_w
    ch = 256
    n_ch = per_w // ch
    nbuf = 3
    ahead = 1
    ntask = 2 * n_ch
    mesh = plsc.VectorSubcoreMesh(core_axis_name="c", subcore_axis_name="s")

    @functools.partial(
        pl.kernel,
        mesh=mesh,
        out_type=[
            jax.ShapeDtypeStruct((_B, 1, _S, _D), jnp.float32),
            jax.ShapeDtypeStruct((_B, 1, _S, _D), jnp.float32),
        ],
        scratch_types=[
            pltpu.VMEM((per_w,), jnp.int32),
        ]
        + [pltpu.VMEM((ch, _D), jnp.float32) for _ in range(nbuf)]
        + [pltpu.SemaphoreType.DMA for _ in range(2 * nbuf)],
    )
    def k(cos_hbm, sin_hbm, idx_hbm, cos_out, sin_out, idx_v, *bufs_sems):
        bufs = bufs_sems[:nbuf]
        gsem = bufs_sems[nbuf:2 * nbuf]
        wsem = bufs_sems[2 * nbuf:]
        wid = lax.axis_index("s") * info.num_cores + lax.axis_index("c")
        b = wid // w_per_b
        s0 = (wid % w_per_b) * per_w

        def task(t):
            c = t // 2
            tab = cos_hbm if t % 2 == 0 else sin_hbm
            out = cos_out if t % 2 == 0 else sin_out
            return tab, out, pl.ds(c * ch, ch), pl.ds(s0 + c * ch, ch)

        def gcopy(t, bi):
            tab, _, in_sl, _ = task(t)
            return pltpu.make_async_copy(tab.at[idx_v.at[in_sl]], bufs[bi],
                                         gsem[bi])

        def wcopy(t, bi):
            _, out, _, out_sl = task(t)
            return pltpu.make_async_copy(bufs[bi], out.at[b, 0, out_sl, :],
                                         wsem[bi])

        pltpu.sync_copy(idx_hbm.at[b, pl.ds(s0, per_w)], idx_v)
        for t in range(ahead):
            gcopy(t, t % nbuf).start()
        for t in range(ntask):
            bi = t % nbuf
            gcopy(t, bi).wait()
            wcopy(t, bi).start()
            nt = t + ahead
            if nt < ntask:
                nb = nt % nbuf
                if nt >= nbuf:
                    wcopy(nt - nbuf, nb).wait()
                gcopy(nt, nb).start()
        for t in range(ntask - nbuf, ntask):
            wcopy(t, t % nbuf).wait()

    return k


@jax.jit
def kernel(x, position_ids, cos_cached, sin_cached):
    idx = position_ids.astype(jnp.int32)
    cos, sin = _gather_kernel()(cos_cached, sin_cached, idx)
    return (cos, sin)


# speedup vs baseline: 1.0413x; 1.0413x over previous
import functools

import jax
import jax.numpy as jnp
from jax import lax
from jax.experimental import pallas as pl
from jax.experimental.pallas import tpu as pltpu
from jax.experimental.pallas import tpu_sc as plsc

_B = 4
_S = 4096
_D = 128
_N = _B * _S


@functools.cache
def _gather_kernel():
    info = plsc.get_sparse_core_info()
    nw = info.num_cores * info.num_subcores
    per_w = _N // nw
    w_per_b = _S // per_w
    ch = 128
    n_ch = per_w // ch
    nbuf = 6
    ahead = 3
    ntask = 2 * n_ch
    mesh = plsc.VectorSubcoreMesh(core_axis_name="c", subcore_axis_name="s")

    @functools.partial(
        pl.kernel,
        mesh=mesh,
        out_type=[
            jax.ShapeDtypeStruct((_B, 1, _S, _D), jnp.float32),
            jax.ShapeDtypeStruct((_B, 1, _S, _D), jnp.float32),
        ],
        scratch_types=[
            pltpu.VMEM((per_w,), jnp.int32),
        ]
        + [pltpu.VMEM((ch, _D), jnp.float32) for _ in range(nbuf)]
        + [pltpu.SemaphoreType.DMA for _ in range(2 * nbuf)],
    )
    def k(cos_hbm, sin_hbm, idx_hbm, cos_out, sin_out, idx_v, *bufs_sems):
        bufs = bufs_sems[:nbuf]
        gsem = bufs_sems[nbuf:2 * nbuf]
        wsem = bufs_sems[2 * nbuf:]
        wid = lax.axis_index("s") * info.num_cores + lax.axis_index("c")
        b = wid // w_per_b
        s0 = (wid % w_per_b) * per_w

        def task(t):
            c = t // 2
            tab = cos_hbm if t % 2 == 0 else sin_hbm
            out = cos_out if t % 2 == 0 else sin_out
            return tab, out, pl.ds(c * ch, ch), pl.ds(s0 + c * ch, ch)

        def gcopy(t, bi):
            tab, _, in_sl, _ = task(t)
            return pltpu.make_async_copy(tab.at[idx_v.at[in_sl]], bufs[bi],
                                         gsem[bi])

        def wcopy(t, bi):
            _, out, _, out_sl = task(t)
            return pltpu.make_async_copy(bufs[bi], out.at[b, 0, out_sl, :],
                                         wsem[bi])

        pltpu.sync_copy(idx_hbm.at[b, pl.ds(s0, per_w)], idx_v)
        for t in range(ahead):
            gcopy(t, t % nbuf).start()
        for t in range(ntask):
            bi = t % nbuf
            gcopy(t, bi).wait()
            wcopy(t, bi).start()
            nt = t + ahead
            if nt < ntask:
                nb = nt % nbuf
                if nt >= nbuf:
                    wcopy(nt - nbuf, nb).wait()
                gcopy(nt, nb).start()
        for t in range(ntask - nbuf, ntask):
            wcopy(t, t % nbuf).wait()

    return k


@jax.jit
def kernel(x, position_ids, cos_cached, sin_cached):
    idx = position_ids.astype(jnp.int32)
    cos, sin = _gather_kernel()(cos_cached, sin_cached, idx)
    return (cos, sin)
